# baseline (speedup 1.0000x reference)
import functools

import jax
import jax.numpy as jnp
from jax import lax
from jax.experimental import pallas as pl
from jax.experimental.pallas import tpu as pltpu
from jax.experimental.pallas import tpu_sc as plsc

ROWS = 4096
COLS = 32768
KTOP = 512
NC = 2
NS = 16
L = 16
NW = NC * NS
NVEC = COLS // L
RPW = ROWS // NW


def _topk_body(x_hbm, out_hbm, row_a, row_b, cand, hist, cnt,
               sem_ia, sem_ib, sem_oa, sem_ob):
  cid = lax.axis_index("c")
  sid = lax.axis_index("s")
  wid = sid * NC + cid

  zeros16 = jnp.zeros((L,), jnp.int32)
  ones16 = jnp.ones((L,), jnp.int32)
  lanes = lax.iota(jnp.int32, L)

  @plsc.parallel_loop(0, 256, unroll=4)
  def _(b):
    hist[pl.ds(b * L, L)] = zeros16

  def threshold(row_buf):
    @plsc.parallel_loop(0, NVEC, unroll=16)
    def _(i):
      v = row_buf[pl.ds(i * L, L)]
      bu = plsc.bitcast(v, jnp.int32)
      idx = (lax.shift_right_logical(bu, 20) & jnp.int32(0xFF0)) | lanes
      plsc.addupdate_scatter(hist, [idx], ones16)

    @plsc.parallel_loop(0, 256, unroll=8)
    def _(b):
      cnt[b] = jnp.sum(hist[pl.ds(b * L, L)])
      hist[pl.ds(b * L, L)] = zeros16

    def cond1(st):
      k, acc = st
      b = jnp.where(k < 128, 127 - k, k)
      return acc + cnt[b] < KTOP

    def body1(st):
      k, acc = st
      b = jnp.where(k < 128, 127 - k, k)
      return k + 1, acc + cnt[b]

    kstar, acc_above = lax.while_loop(cond1, body1,
                                      (jnp.int32(0), jnp.int32(0)))
    b1raw = jnp.where(kstar < 128, 127 - kstar, kstar)
    rank = jnp.int32(KTOP) - acc_above
    prefix = b1raw

    @plsc.parallel_loop(0, NVEC, unroll=8, carry=jnp.int32(0))
    def ncand(i, off):
      v = row_buf[pl.ds(i * L, L)]
      bu = plsc.bitcast(v, jnp.int32)
      raw = lax.shift_right_logical(bu, 24)
      m = raw == b1raw
      plsc.store_compressed(cand.at[pl.ds(off, L)], bu, mask=m)
      return off + plsc.all_reduce_population_count(m)[0]

    nv = (ncand + (L - 1)) // L
    is_pos = b1raw < 128

    def round_body(ri, st):
      prefix, rank = st
      sh = 20 - 4 * ri

      @plsc.parallel_loop(0, nv, unroll=2)
      def _(i):
        s = cand[pl.ds(i * L, L)]
        valid = (i * L + lanes) < ncand
        m = valid & (lax.shift_right_logical(s, sh + 4) == prefix)
        idx = ((lax.shift_right_logical(s, sh) & jnp.int32(0xF)) * L) | lanes
        plsc.addupdate_scatter(hist, [idx], ones16, mask=m)

      for b in range(16):
        cnt[b] = jnp.sum(hist[pl.ds(b * L, L)])
        hist[pl.ds(b * L, L)] = zeros16

      def cond(cs_):
        w, a = cs_
        b = jnp.where(is_pos, 15 - w, w)
        return a + cnt[b] < rank

      def body(cs_):
        w, a = cs_
        b = jnp.where(is_pos, 15 - w, w)
        return w + 1, a + cnt[b]

      wstar, acc_ab = lax.while_loop(cond, body,
                                     (jnp.int32(0), jnp.int32(0)))
      bstar = jnp.where(is_pos, 15 - wstar, wstar)
      return prefix * 16 + bstar, rank - acc_ab

    prefix, rank = lax.fori_loop(0, 6, round_body, (prefix, rank))
    t_key = jnp.where(prefix < 0, prefix ^ jnp.int32(0x7FFFFFFF), prefix)
    return jnp.maximum(t_key, 1)

  def mask_row(row_buf, t):
    @plsc.parallel_loop(0, NVEC, unroll=16)
    def _(i):
      v = row_buf[pl.ds(i * L, L)]
      bu = plsc.bitcast(v, jnp.int32)
      row_buf[pl.ds(i * L, L)] = jnp.where(bu >= t, v, jnp.float32(0.0))

  pltpu.async_copy(x_hbm.at[wid], row_a, sem_ia)

  def do_pair(jj, carry):
    r0 = wid + (2 * jj) * NW
    r1 = r0 + NW
    pltpu.make_async_copy(x_hbm.at[r0], row_a, sem_ia).wait()

    @pl.when(jj > 0)
    def _():
      pltpu.make_async_copy(row_b, out_hbm.at[r1 - 2 * NW], sem_ob).wait()

    pltpu.async_copy(x_hbm.at[r1], row_b, sem_ib)

    t0 = threshold(row_a)
    mask_row(row_a, t0)
    pltpu.async_copy(row_a, out_hbm.at[r0], sem_oa)

    pltpu.make_async_copy(x_hbm.at[r1], row_b, sem_ib).wait()
    t1 = threshold(row_b)

    pltpu.make_async_copy(row_a, out_hbm.at[r0], sem_oa).wait()

    @pl.when(jj < RPW // 2 - 1)
    def _():
      pltpu.async_copy(x_hbm.at[r0 + 2 * NW], row_a, sem_ia)

    mask_row(row_b, t1)
    pltpu.async_copy(row_b, out_hbm.at[r1], sem_ob)
    return carry

  lax.fori_loop(0, RPW // 2, do_pair, 0)
  last_r1 = wid + (RPW - 1) * NW
  pltpu.make_async_copy(row_b, out_hbm.at[last_r1], sem_ob).wait()


kernel = functools.partial(
    pl.kernel,
    out_type=jax.ShapeDtypeStruct((ROWS, COLS), jnp.float32),
    mesh=plsc.VectorSubcoreMesh(
        core_axis_name="c", subcore_axis_name="s",
        num_cores=NC, num_subcores=NS),
    scratch_types=[
        pltpu.VMEM((COLS,), jnp.float32),
        pltpu.VMEM((COLS,), jnp.float32),
        pltpu.VMEM((COLS,), jnp.int32),
        pltpu.VMEM((256 * L,), jnp.int32),
        pltpu.SMEM((256,), jnp.int32),
        pltpu.SemaphoreType.DMA,
        pltpu.SemaphoreType.DMA,
        pltpu.SemaphoreType.DMA,
        pltpu.SemaphoreType.DMA,
    ],
    compiler_params=pltpu.CompilerParams(needs_layout_passes=False),
)(_topk_body)
